# baseline (speedup 1.0000x reference)
import jax
import jax.numpy as jnp
from jax import lax
from jax.experimental import pallas as pl
from jax.experimental.pallas import tpu as pltpu
from jax.experimental.pallas import tpu_sc as plsc

NC = 2
NS = 16
NW = NC * NS

D = 64
L = 200
B = 4096
CB = B // NW
DT = D // 8
BT = B // 128


def _body(idx_hbm, tok_hbm, pos_hbm, out_hbm, idx_v, pos_v,
          b0, b1, t0, t1, g0, g1, s0, s1):
  bufs = (b0, b1)
  stages = (t0, t1)
  gsems = (g0, g1)
  ssems = (s0, s1)

  wid = lax.axis_index("s") * NC + lax.axis_index("c")
  col0 = wid * CB

  pltpu.sync_copy(idx_hbm.at[:, pl.ds(col0, CB)], idx_v)
  pltpu.sync_copy(pos_hbm, pos_v)

  def start_gather(l, p):
    pltpu.async_copy(tok_hbm.at[idx_v.at[l]], bufs[p], gsems[p])

  def wait_gather(l, p):
    pltpu.make_async_copy(tok_hbm.at[idx_v.at[l]], bufs[p], gsems[p]).wait()

  def start_store(l, p):
    pltpu.async_copy(stages[p], out_hbm.at[l, :, wid], ssems[p])

  def wait_store(l, p):
    pltpu.make_async_copy(
        stages[p], out_hbm.at[l, :, wid], ssems[p]).wait()

  def transpose_add(l, p):
    buf = bufs[p]
    stage = stages[p]
    lvec = jnp.full((16,), l, jnp.int32)
    rows_g = tuple(g * 16 + lax.iota(jnp.int32, 16)
                   for g in range(CB // 16))

    @plsc.parallel_loop(0, DT)
    def dloop(dt):
      d0 = dt * 8
      for dr in range(8):
        cols = jnp.full((16,), d0 + dr, jnp.int32)
        ps = plsc.load_gather(pos_v, [lvec, cols])
        for g in range(CB // 16):
          gath = plsc.load_gather(buf, [rows_g[g], cols])
          stage[dt, dr, pl.ds(g * 16, 16)] = gath + ps

  start_gather(0, 0)

  start_gather(1, 1)
  wait_gather(0, 0)
  transpose_add(0, 0)
  start_store(0, 0)

  start_gather(2, 0)
  wait_gather(1, 1)
  transpose_add(1, 1)
  start_store(1, 1)

  def pair(g, carry):
    for u_off in range(2):
      l = 2 * g + 2 + u_off
      p = u_off
      start_gather(l + 1, 1 - p)
      wait_gather(l, p)
      wait_store(l - 2, p)
      transpose_add(l, p)
      start_store(l, p)
    return carry

  lax.fori_loop(0, (L - 4) // 2, pair, 0)

  start_gather(199, 1)
  wait_gather(198, 0)
  wait_store(196, 0)
  transpose_add(198, 0)
  start_store(198, 0)

  wait_gather(199, 1)
  wait_store(197, 1)
  transpose_add(199, 1)
  start_store(199, 1)

  wait_store(198, 0)
  wait_store(199, 1)


@jax.jit
def _run(idxT, tok2, pos):
  mesh = plsc.VectorSubcoreMesh(
      core_axis_name="c", subcore_axis_name="s", num_cores=NC,
      num_subcores=NS)
  f = pl.kernel(
      _body,
      out_type=jax.ShapeDtypeStruct((L, DT, BT, 8, 128), jnp.float32),
      mesh=mesh,
      scratch_types=[
          pltpu.VMEM((L, CB), jnp.int32),
          pltpu.VMEM((L, D), jnp.float32),
          pltpu.VMEM((CB, 2 * D), jnp.float32),
          pltpu.VMEM((CB, 2 * D), jnp.float32),
          pltpu.VMEM((DT, 8, 128), jnp.float32),
          pltpu.VMEM((DT, 8, 128), jnp.float32),
          pltpu.SemaphoreType.DMA,
          pltpu.SemaphoreType.DMA,
          pltpu.SemaphoreType.DMA,
          pltpu.SemaphoreType.DMA,
      ],
      compiler_params=pltpu.CompilerParams(needs_layout_passes=False),
  )
  return f(idxT, tok2, pos)


def kernel(inputs, token_table, position_table):
  idxT = inputs.astype(jnp.int32).T
  tok2 = jnp.pad(token_table, ((0, 0), (0, D)))
  out5 = _run(idxT, tok2, position_table)
  return out5.transpose(2, 4, 0, 1, 3).reshape(B, L, D)


# speedup vs baseline: 1.1579x; 1.1579x over previous
import functools

import jax
import jax.numpy as jnp
from jax import lax
from jax.experimental import pallas as pl
from jax.experimental.pallas import tpu as pltpu
from jax.experimental.pallas import tpu_sc as plsc

NC = 2
NS = 16
NW = NC * NS

D = 64
L = 200
B = 4096
TOTAL = B * L
CHUNK = 128
PER_W = TOTAL // NW
N_CHUNKS = PER_W // CHUNK
NBUF = 3
HL = L // 2
POS_EXT = HL + CHUNK // 2


def _body(idx_hbm, tok_hbm, pos_hbm, out_hbm, idx_v, pos_v,
          b0, b1, b2, h0, h1, h2, g0, g1, g2, s0, s1, s2):
  bufs = (b0, b1, b2)
  idxcs = (h0, h1, h2)
  gsems = (g0, g1, g2)
  ssems = (s0, s1, s2)

  wid = lax.axis_index("s") * NC + lax.axis_index("c")
  crow0 = wid * N_CHUNKS
  row0 = wid * PER_W

  pltpu.sync_copy(idx_hbm.at[pl.ds(crow0, N_CHUNKS)], idx_v)
  pltpu.sync_copy(pos_hbm, pos_v.at[pl.ds(0, HL)])
  pltpu.sync_copy(pos_hbm.at[pl.ds(0, POS_EXT - HL)],
                  pos_v.at[pl.ds(HL, POS_EXT - HL)])

  def start_gather(k, p):
    for g in range(CHUNK // 16):
      s = pl.ds(g * 16, 16)
      idxcs[p][s] = idx_v[k, s]
    pltpu.async_copy(tok_hbm.at[idxcs[p]], bufs[p], gsems[p])

  def wait_gather(p):
    pltpu.make_async_copy(tok_hbm.at[idxcs[p]], bufs[p], gsems[p]).wait()

  def start_store(k, p):
    dst = out_hbm.at[pl.ds(row0 + k * CHUNK, CHUNK)]
    pltpu.async_copy(bufs[p], dst, ssems[p])

  def wait_store(k, p):
    dst = out_hbm.at[pl.ds(row0 + k * CHUNK, CHUNK)]
    pltpu.make_async_copy(bufs[p], dst, ssems[p]).wait()

  def add_pos(k, p):
    buf = bufs[p]
    phase2 = lax.rem(k * (CHUNK // 2), HL)

    @plsc.parallel_loop(0, CHUNK // 16, unroll=2)
    def rows(g):
      for u in range(16):
        r = g * 16 + u
        pr = phase2 + g * 8 + u // 2
        for v in range(D // 16):
          so = pl.ds(v * 16, 16)
          po = pl.ds((u % 2) * D + v * 16, 16)
          buf[r, so] = buf[r, so] + pos_v[pr, po]

  start_gather(0, 0)
  start_gather(1, 1)

  start_gather(2, 2)
  wait_gather(0)
  add_pos(0, 0)
  start_store(0, 0)

  def group(g, carry):
    for p_off in range(NBUF):
      k = NBUF * g + 1 + p_off
      pc = (1 + p_off) % NBUF
      pn = (p_off) % NBUF
      wait_store(k - 1, pn)
      start_gather(k + 2, pn)
      wait_gather(pc)
      add_pos(k, pc)
      start_store(k, pc)
    return carry

  lax.fori_loop(0, (N_CHUNKS - 5) // NBUF, group, 0)

  for k in range(N_CHUNKS - 4, N_CHUNKS):
    pc = k % NBUF
    pn = (k - 1) % NBUF
    wait_store(k - 1, pn)
    if k + 2 < N_CHUNKS:
      start_gather(k + 2, pn)
    wait_gather(pc)
    add_pos(k, pc)
    start_store(k, pc)
  wait_store(N_CHUNKS - 1, (N_CHUNKS - 1) % NBUF)


@jax.jit
def _run(idx, tok2, pos2):
  mesh = plsc.VectorSubcoreMesh(
      core_axis_name="c", subcore_axis_name="s", num_cores=NC,
      num_subcores=NS)
  f = pl.kernel(
      _body,
      out_type=jax.ShapeDtypeStruct((TOTAL, 2 * D), jnp.float32),
      mesh=mesh,
      scratch_types=[
          pltpu.VMEM((N_CHUNKS, CHUNK), jnp.int32),
          pltpu.VMEM((POS_EXT, 2 * D), jnp.float32),
          pltpu.VMEM((CHUNK, 2 * D), jnp.float32),
          pltpu.VMEM((CHUNK, 2 * D), jnp.float32),
          pltpu.VMEM((CHUNK, 2 * D), jnp.float32),
          pltpu.VMEM((CHUNK,), jnp.int32),
          pltpu.VMEM((CHUNK,), jnp.int32),
          pltpu.VMEM((CHUNK,), jnp.int32),
          pltpu.SemaphoreType.DMA,
          pltpu.SemaphoreType.DMA,
          pltpu.SemaphoreType.DMA,
          pltpu.SemaphoreType.DMA,
          pltpu.SemaphoreType.DMA,
          pltpu.SemaphoreType.DMA,
      ],
  )
  return f(idx, tok2, pos2)


def kernel(inputs, token_table, position_table):
  idx = inputs.astype(jnp.int32).reshape(TOTAL // CHUNK, CHUNK)
  tok2 = jnp.pad(token_table, ((0, 0), (0, D)))
  pos2 = position_table.reshape(HL, 2 * D)
  out2 = _run(idx, tok2, pos2)
  return out2[:, :D].reshape(B, L, D)
